# Initial kernel scaffold; baseline (speedup 1.0000x reference)
import jax
import jax.numpy as jnp
from jax.experimental import pallas as pl


def kernel(edge_index, edge_type, emb, basis1, comp1, root1, bias1, basis2, comp2, root2, bias2):
    raise NotImplementedError("write your pallas kernel here")


# speedup vs baseline: 25.8748x; 25.8748x over previous
import functools

import jax
import jax.numpy as jnp
from jax import lax
from jax.experimental import pallas as pl
from jax.experimental.pallas import tpu as pltpu
from jax.experimental.pallas import tpu_sc as plsc

N = 10000
R = 16
NBASES = 16
H = 128
E = 320000
NC, NS, LANES = 2, 16, 16
NW = NC * NS
CH = 79
EPT = CH * 128
EP = NW * EPT
RN = R * N
CNT_PER_TILE = 10112
CNTP = NS * CNT_PER_TILE
NP = 10240
BN = 1000


def _sc_count(key2_hbm, cnt_out, cnt_sh, key2_v, zeros_v, ones_v):
    cid = lax.axis_index("c")
    sid = lax.axis_index("s")
    wid = cid * NS + sid
    zv = jnp.zeros((LANES,), jnp.float32)

    def zfill(i, _):
        zeros_v[pl.ds(i * LANES, LANES)] = zv
        return 0

    lax.fori_loop(0, CNT_PER_TILE // LANES, zfill, 0)
    for k in range(128 // LANES):
        ones_v[pl.ds(k * LANES, LANES)] = jnp.ones((LANES,), jnp.float32)
    pltpu.sync_copy(zeros_v, cnt_sh.at[pl.ds(sid * CNT_PER_TILE, CNT_PER_TILE)])
    pltpu.sync_copy(key2_hbm.at[wid], key2_v)
    plsc.subcore_barrier()

    def body(j, _):
        pltpu.sync_copy(ones_v, cnt_sh.at[key2_v.at[j]], add=True)
        return 0

    lax.fori_loop(0, CH, body, 0)
    plsc.subcore_barrier()
    pltpu.sync_copy(
        cnt_sh.at[pl.ds(sid * CNT_PER_TILE, CNT_PER_TILE)],
        cnt_out.at[pl.ds(cid * CNTP + sid * CNT_PER_TILE, CNT_PER_TILE)],
    )


def _sc_weights(c0_hbm, c1_hbm, key2_hbm, w_out, key2_v, w_v, c0_v, c1_v, sem0, sem1):
    cid = lax.axis_index("c")
    sid = lax.axis_index("s")
    wid = cid * NS + sid
    pltpu.sync_copy(key2_hbm.at[wid], key2_v)

    def body(j, _):
        cp0 = pltpu.async_copy(c0_hbm.at[key2_v.at[j]], c0_v, sem0)
        cp1 = pltpu.async_copy(c1_hbm.at[key2_v.at[j]], c1_v, sem1)
        cp0.wait()
        cp1.wait()
        for k in range(128 // LANES):
            sl = pl.ds(k * LANES, LANES)
            c = c0_v[sl] + c1_v[sl]
            w_v[j, sl] = 1.0 / jnp.maximum(c, 1.0)
        return 0

    lax.fori_loop(0, CH, body, 0)
    pltpu.sync_copy(w_v, w_out.at[wid])


def _sc_agg(y_hbm, key1_hbm, w_hbm, dst_hbm, part_out,
            acc_sh, key1_v, w_v, dst_v, rows_v, sem):
    cid = lax.axis_index("c")
    sid = lax.axis_index("s")
    wid = cid * NS + sid
    zv = jnp.zeros((LANES,), jnp.float32)

    def zfill(i, _):
        for k in range(H // LANES):
            rows_v[i, pl.ds(k * LANES, LANES)] = zv
        return 0

    lax.fori_loop(0, 128, zfill, 0)
    for k in range(5):
        pltpu.sync_copy(rows_v, acc_sh.at[pl.ds(sid * 640 + k * 128, 128)])
    pltpu.sync_copy(key1_hbm.at[wid], key1_v)
    pltpu.sync_copy(w_hbm.at[wid], w_v)
    pltpu.sync_copy(dst_hbm.at[wid], dst_v)
    plsc.subcore_barrier()

    def body(j, _):
        pltpu.async_copy(y_hbm.at[key1_v.at[j]], rows_v, sem).wait()

        def scale(i, _):
            wsp = plsc.load_gather(
                w_v,
                [jnp.full((LANES,), j, jnp.int32), jnp.full((LANES,), i, jnp.int32)],
            )
            for k in range(H // LANES):
                sl = pl.ds(k * LANES, LANES)
                rows_v[i, sl] = rows_v[i, sl] * wsp
            return 0

        lax.fori_loop(0, 128, scale, 0)
        pltpu.sync_copy(rows_v, acc_sh.at[dst_v.at[j]], add=True)
        return 0

    lax.fori_loop(0, CH, body, 0)
    plsc.subcore_barrier()
    for k in range(5):
        sl = pl.ds(sid * 640 + k * 128, 128)
        pltpu.sync_copy(acc_sh.at[sl], part_out.at[cid, sl])


def _tc_weights_body(comp_ref, basis_ref, w_ref):
    w_ref[...] = jnp.dot(
        comp_ref[...], basis_ref[...], preferred_element_type=jnp.float32
    )


def _tc_transform_body(w_ref, x_ref, y_ref):
    y_ref[...] = jnp.dot(
        x_ref[...], w_ref[0], preferred_element_type=jnp.float32
    )[None]


def _tc_combine_body(x_ref, root_ref, bias_ref, p_ref, o_ref, *, act):
    t = jnp.dot(x_ref[...], root_ref[...], preferred_element_type=jnp.float32)
    t = t + bias_ref[...] + p_ref[0] + p_ref[1]
    o_ref[...] = jnp.maximum(t, 0.0) if act else t


def _transform(comp, basis, x):
    wmat = pl.pallas_call(
        _tc_weights_body,
        in_specs=[
            pl.BlockSpec((R, NBASES), lambda: (0, 0)),
            pl.BlockSpec((NBASES, H * H), lambda: (0, 0)),
        ],
        out_specs=pl.BlockSpec((R, H * H), lambda: (0, 0)),
        out_shape=jax.ShapeDtypeStruct((R, H * H), jnp.float32),
    )(comp, basis.reshape(NBASES, H * H))
    return pl.pallas_call(
        _tc_transform_body,
        grid=(R, N // BN),
        in_specs=[
            pl.BlockSpec((1, H, H), lambda r, i: (r, 0, 0)),
            pl.BlockSpec((BN, H), lambda r, i: (i, 0)),
        ],
        out_specs=pl.BlockSpec((1, BN, H), lambda r, i: (r, i, 0)),
        out_shape=jax.ShapeDtypeStruct((R, N, H), jnp.float32),
    )(wmat.reshape(R, H, H), x)


def _combine(x, root, bias, part, act):
    return pl.pallas_call(
        functools.partial(_tc_combine_body, act=act),
        grid=(N // BN,),
        in_specs=[
            pl.BlockSpec((BN, H), lambda i: (i, 0)),
            pl.BlockSpec((H, H), lambda i: (0, 0)),
            pl.BlockSpec((1, H), lambda i: (0, 0)),
            pl.BlockSpec((2, BN, H), lambda i: (0, i, 0)),
        ],
        out_specs=pl.BlockSpec((BN, H), lambda i: (i, 0)),
        out_shape=jax.ShapeDtypeStruct((N, H), jnp.float32),
    )(x, root, bias, part)


_sc_mesh = plsc.VectorSubcoreMesh(core_axis_name="c", subcore_axis_name="s")
_sc_params = pltpu.CompilerParams(needs_layout_passes=False)

_count_call = pl.kernel(
    _sc_count,
    out_type=jax.ShapeDtypeStruct((NC * CNTP,), jnp.float32),
    mesh=_sc_mesh,
    compiler_params=_sc_params,
    scratch_types=[
        pltpu.VMEM_SHARED((CNTP,), jnp.float32),
        pltpu.VMEM((CH, 128), jnp.int32),
        pltpu.VMEM((CNT_PER_TILE,), jnp.float32),
        pltpu.VMEM((128,), jnp.float32),
    ],
)

_weights_call = pl.kernel(
    _sc_weights,
    out_type=jax.ShapeDtypeStruct((NW, CH, 128), jnp.float32),
    mesh=_sc_mesh,
    compiler_params=_sc_params,
    scratch_types=[
        pltpu.VMEM((CH, 128), jnp.int32),
        pltpu.VMEM((CH, 128), jnp.float32),
        pltpu.VMEM((128,), jnp.float32),
        pltpu.VMEM((128,), jnp.float32),
        pltpu.SemaphoreType.DMA,
        pltpu.SemaphoreType.DMA,
    ],
)

_agg_call = pl.kernel(
    _sc_agg,
    out_type=jax.ShapeDtypeStruct((NC, NP, H), jnp.float32),
    mesh=_sc_mesh,
    compiler_params=_sc_params,
    scratch_types=[
        pltpu.VMEM_SHARED((NP, H), jnp.float32),
        pltpu.VMEM((CH, 128), jnp.int32),
        pltpu.VMEM((CH, 128), jnp.float32),
        pltpu.VMEM((CH, 128), jnp.int32),
        pltpu.VMEM((128, H), jnp.float32),
        pltpu.SemaphoreType.DMA,
    ],
)


def kernel(edge_index, edge_type, emb, basis1, comp1, root1, bias1,
           basis2, comp2, root2, bias2):
    src = edge_index[0].astype(jnp.int32)
    dst = edge_index[1].astype(jnp.int32)
    et = edge_type.astype(jnp.int32)
    pad = EP - E
    key1 = jnp.concatenate([et * N + src, jnp.zeros((pad,), jnp.int32)])
    key2 = jnp.concatenate([et * N + dst, jnp.full((pad,), RN, jnp.int32)])
    dstp = jnp.concatenate([dst, jnp.full((pad,), N, jnp.int32)])
    key1 = key1.reshape(NW, CH, 128)
    key2 = key2.reshape(NW, CH, 128)
    dstp = dstp.reshape(NW, CH, 128)

    cnt = _count_call(key2)
    w = _weights_call(cnt[:CNTP], cnt[CNTP:], key2)

    y1 = _transform(comp1, basis1, emb)
    p1 = _agg_call(y1.reshape(RN, H), key1, w, dstp)
    x1 = _combine(emb, root1, bias1.reshape(1, H), p1, True)

    y2 = _transform(comp2, basis2, x1)
    p2 = _agg_call(y2.reshape(RN, H), key1, w, dstp)
    out = _combine(x1, root2, bias2.reshape(1, H), p2, False)
    return out


# speedup vs baseline: 36.2404x; 1.4006x over previous
import functools

import jax
import jax.numpy as jnp
from jax import lax
from jax.experimental import pallas as pl
from jax.experimental.pallas import tpu as pltpu
from jax.experimental.pallas import tpu_sc as plsc

N = 10000
R = 16
NBASES = 16
H = 128
E = 320000
NC, NS, LANES = 2, 16, 16
NW = NC * NS
CR = 80
CH = 126
G = 21
NG = CH // G
EPT = CH * CR
EP = NW * EPT
RN = R * N
CNT_PER_TILE = 10112
CNTP = NS * CNT_PER_TILE
NP = 10112
TPR = NP // NS
NZC = TPR // CR
BN = 1000


def _sc_count(key2_hbm, cnt_out, cnt_sh, key2_v, zeros_v, ones_v):
    cid = lax.axis_index("c")
    sid = lax.axis_index("s")
    wid = cid * NS + sid
    zv = jnp.zeros((LANES,), jnp.float32)

    def zfill(i, _):
        zeros_v[pl.ds(i * LANES, LANES)] = zv
        return 0

    lax.fori_loop(0, CNT_PER_TILE // LANES, zfill, 0)
    for k in range(CR // LANES):
        ones_v[pl.ds(k * LANES, LANES)] = jnp.ones((LANES,), jnp.float32)
    pltpu.sync_copy(zeros_v, cnt_sh.at[pl.ds(sid * CNT_PER_TILE, CNT_PER_TILE)])
    pltpu.sync_copy(key2_hbm.at[wid], key2_v)
    plsc.subcore_barrier()

    def body(j, _):
        pltpu.sync_copy(ones_v, cnt_sh.at[key2_v.at[j]], add=True)
        return 0

    lax.fori_loop(0, CH, body, 0)
    plsc.subcore_barrier()
    pltpu.sync_copy(
        cnt_sh.at[pl.ds(sid * CNT_PER_TILE, CNT_PER_TILE)],
        cnt_out.at[pl.ds(cid * CNTP + sid * CNT_PER_TILE, CNT_PER_TILE)],
    )


def _sc_weights(c0_hbm, c1_hbm, key2_hbm, w_out, key2_v, w_v, c0_v, c1_v, sem0, sem1):
    cid = lax.axis_index("c")
    sid = lax.axis_index("s")
    wid = cid * NS + sid
    pltpu.sync_copy(key2_hbm.at[wid], key2_v)

    def body(j, _):
        cp0 = pltpu.async_copy(c0_hbm.at[key2_v.at[j]], c0_v, sem0)
        cp1 = pltpu.async_copy(c1_hbm.at[key2_v.at[j]], c1_v, sem1)
        cp0.wait()
        cp1.wait()
        for k in range(CR // LANES):
            sl = pl.ds(k * LANES, LANES)
            c = c0_v[sl] + c1_v[sl]
            w_v[j, sl] = 1.0 / jnp.maximum(c, 1.0)
        return 0

    lax.fori_loop(0, CH, body, 0)
    pltpu.sync_copy(w_v, w_out.at[wid])


def _sc_agg(y_hbm, key1_hbm, dst_hbm, w_hbm, part_out,
            acc_sh, k1buf, dstbuf, wbuf, rows, gsems, ssems, ksem):
    cid = lax.axis_index("c")
    sid = lax.axis_index("s")
    wid = cid * NS + sid
    zv = jnp.zeros((LANES,), jnp.float32)

    def zfill(i, _):
        for k in range(H // LANES):
            rows[0][i, pl.ds(k * LANES, LANES)] = zv
        return 0

    lax.fori_loop(0, CR, zfill, 0)
    base_r = sid * TPR
    for i in range(NZC):
        pltpu.sync_copy(rows[0], acc_sh.at[pl.ds(base_r + i * CR, CR)])
    rem = TPR - NZC * CR
    pltpu.sync_copy(rows[0].at[pl.ds(0, rem)],
                    acc_sh.at[pl.ds(base_r + NZC * CR, rem)])
    plsc.subcore_barrier()

    def kload(g, p):
        pltpu.async_copy(key1_hbm.at[wid, g], k1buf[p], ksem)
        pltpu.async_copy(dst_hbm.at[wid, g], dstbuf[p], ksem)
        pltpu.async_copy(w_hbm.at[wid, g], wbuf[p], ksem)

    def kwait(p):
        pltpu.make_async_copy(key1_hbm.at[wid, 0], k1buf[p], ksem).wait()
        pltpu.make_async_copy(dst_hbm.at[wid, 0], dstbuf[p], ksem).wait()
        pltpu.make_async_copy(w_hbm.at[wid, 0], wbuf[p], ksem).wait()

    def scale(c, p, buf):
        cvec = jnp.full((LANES,), c, jnp.int32)

        def grp(q, _):
            base = q * 4
            bvec = jnp.full((LANES,), base, jnp.int32)
            for r in range(4):
                wsp = plsc.load_gather(wbuf[p], [cvec, bvec + r])
                row = base + r
                for k in range(H // LANES):
                    sl = pl.ds(k * LANES, LANES)
                    buf[row, sl] = buf[row, sl] * wsp
            return 0

        lax.fori_loop(0, CR // 4, grp, 0)

    def chunk(c, b, p):
        bn = (b + 1) % 3
        pltpu.make_async_copy(rows[bn], acc_sh.at[dstbuf[p].at[c - 2]],
                              ssems[bn]).wait()
        pltpu.async_copy(y_hbm.at[k1buf[p].at[c + 1]], rows[bn], gsems[bn])
        pltpu.make_async_copy(y_hbm.at[k1buf[p].at[c]], rows[b], gsems[b]).wait()
        scale(c, p, rows[b])
        pltpu.async_copy(rows[b], acc_sh.at[dstbuf[p].at[c]], ssems[b], add=True)

    def last_chunk(c, b, p):
        bn = (b + 1) % 3
        pltpu.make_async_copy(rows[bn], acc_sh.at[dstbuf[p].at[c - 2]],
                              ssems[bn]).wait()
        pltpu.make_async_copy(y_hbm.at[k1buf[p].at[c]], rows[b], gsems[b]).wait()
        scale(c, p, rows[b])
        pltpu.async_copy(rows[b], acc_sh.at[dstbuf[p].at[c]], ssems[b], add=True)

    kload(0, 0)
    kwait(0)
    pltpu.async_copy(y_hbm.at[k1buf[0].at[0]], rows[0], gsems[0])

    def pair(g2, _):
        for half in range(2):
            g = g2 * 2 + half
            p = half
            for b in range(3):
                bn = (b + 1) % 3
                if half == 0 and b < 2:
                    @pl.when(g2 > 0)
                    def _(bn=bn, b=b):
                        pltpu.make_async_copy(
                            rows[bn], acc_sh.at[dstbuf[p].at[b]], ssems[bn]).wait()
                else:
                    pltpu.make_async_copy(
                        rows[bn], acc_sh.at[dstbuf[p].at[b]], ssems[bn]).wait()
                pltpu.async_copy(y_hbm.at[k1buf[p].at[b + 1]], rows[bn], gsems[bn])
                pltpu.make_async_copy(y_hbm.at[k1buf[p].at[b]], rows[b],
                                      gsems[b]).wait()
                scale(b, p, rows[b])
                pltpu.async_copy(rows[b], acc_sh.at[dstbuf[p].at[b]],
                                 ssems[b], add=True)
            if half == 0:
                kload(g + 1, 1 - p)
            else:
                @pl.when(g2 < NG // 2 - 1)
                def _():
                    kload(g + 1, 1 - p)
            def inner(j3, _):
                for b in range(3):
                    c = j3 * 3 + b
                    chunk(c, b, p)
                return 0

            lax.fori_loop(1, G // 3 - 1, inner, 0)
            cb = G - 3
            chunk(cb, cb % 3, p)
            chunk(cb + 1, (cb + 1) % 3, p)
            last_chunk(G - 1, (G - 1) % 3, p)
            if half == 0:
                kwait(1 - p)
                pltpu.async_copy(y_hbm.at[k1buf[1 - p].at[0]], rows[0], gsems[0])
            else:
                @pl.when(g2 < NG // 2 - 1)
                def _():
                    kwait(1 - p)
                    pltpu.async_copy(y_hbm.at[k1buf[1 - p].at[0]], rows[0],
                                     gsems[0])
        return 0

    lax.fori_loop(0, NG // 2, pair, 0)
    pfin = (NG - 1) % 2
    for c in (G - 2, G - 1):
        b = c % 3
        pltpu.make_async_copy(rows[b], acc_sh.at[dstbuf[pfin].at[c]],
                              ssems[b]).wait()
    plsc.subcore_barrier()
    for i in range(NZC):
        sl = pl.ds(base_r + i * CR, CR)
        pltpu.sync_copy(acc_sh.at[sl], part_out.at[cid, sl])
    sl = pl.ds(base_r + NZC * CR, rem)
    pltpu.sync_copy(acc_sh.at[sl], part_out.at[cid, sl])


def _tc_weights_body(comp_ref, basis_ref, w_ref):
    w_ref[...] = jnp.dot(
        comp_ref[...], basis_ref[...], preferred_element_type=jnp.float32
    )


def _tc_transform_body(w_ref, x_ref, y_ref):
    y_ref[...] = jnp.dot(
        x_ref[...], w_ref[0], preferred_element_type=jnp.float32
    )[None]


def _tc_combine_body(x_ref, root_ref, bias_ref, p_ref, o_ref, *, act):
    t = jnp.dot(x_ref[...], root_ref[...], preferred_element_type=jnp.float32)
    t = t + bias_ref[...] + p_ref[0] + p_ref[1]
    o_ref[...] = jnp.maximum(t, 0.0) if act else t


def _transform(comp, basis, x):
    wmat = pl.pallas_call(
        _tc_weights_body,
        in_specs=[
            pl.BlockSpec((R, NBASES), lambda: (0, 0)),
            pl.BlockSpec((NBASES, H * H), lambda: (0, 0)),
        ],
        out_specs=pl.BlockSpec((R, H * H), lambda: (0, 0)),
        out_shape=jax.ShapeDtypeStruct((R, H * H), jnp.float32),
    )(comp, basis.reshape(NBASES, H * H))
    return pl.pallas_call(
        _tc_transform_body,
        grid=(R, N // BN),
        in_specs=[
            pl.BlockSpec((1, H, H), lambda r, i: (r, 0, 0)),
            pl.BlockSpec((BN, H), lambda r, i: (i, 0)),
        ],
        out_specs=pl.BlockSpec((1, BN, H), lambda r, i: (r, i, 0)),
        out_shape=jax.ShapeDtypeStruct((R, N, H), jnp.float32),
    )(wmat.reshape(R, H, H), x)


def _combine(x, root, bias, part, act):
    return pl.pallas_call(
        functools.partial(_tc_combine_body, act=act),
        grid=(N // BN,),
        in_specs=[
            pl.BlockSpec((BN, H), lambda i: (i, 0)),
            pl.BlockSpec((H, H), lambda i: (0, 0)),
            pl.BlockSpec((1, H), lambda i: (0, 0)),
            pl.BlockSpec((2, BN, H), lambda i: (0, i, 0)),
        ],
        out_specs=pl.BlockSpec((BN, H), lambda i: (i, 0)),
        out_shape=jax.ShapeDtypeStruct((N, H), jnp.float32),
    )(x, root, bias, part)


_sc_mesh = plsc.VectorSubcoreMesh(core_axis_name="c", subcore_axis_name="s")
_sc_params = pltpu.CompilerParams(needs_layout_passes=False)

_count_call = pl.kernel(
    _sc_count,
    out_type=jax.ShapeDtypeStruct((NC * CNTP,), jnp.float32),
    mesh=_sc_mesh,
    compiler_params=_sc_params,
    scratch_types=[
        pltpu.VMEM_SHARED((CNTP,), jnp.float32),
        pltpu.VMEM((CH, CR), jnp.int32),
        pltpu.VMEM((CNT_PER_TILE,), jnp.float32),
        pltpu.VMEM((CR,), jnp.float32),
    ],
)

_weights_call = pl.kernel(
    _sc_weights,
    out_type=jax.ShapeDtypeStruct((NW, CH, CR), jnp.float32),
    mesh=_sc_mesh,
    compiler_params=_sc_params,
    scratch_types=[
        pltpu.VMEM((CH, CR), jnp.int32),
        pltpu.VMEM((CH, CR), jnp.float32),
        pltpu.VMEM((CR,), jnp.float32),
        pltpu.VMEM((CR,), jnp.float32),
        pltpu.SemaphoreType.DMA,
        pltpu.SemaphoreType.DMA,
    ],
)

_agg_call = pl.kernel(
    _sc_agg,
    out_type=jax.ShapeDtypeStruct((NC, NP, H), jnp.float32),
    mesh=_sc_mesh,
    compiler_params=_sc_params,
    scratch_types=[
        pltpu.VMEM_SHARED((NP, H), jnp.float32),
        [pltpu.VMEM((G, CR), jnp.int32) for _ in range(2)],
        [pltpu.VMEM((G, CR), jnp.int32) for _ in range(2)],
        [pltpu.VMEM((G, CR), jnp.float32) for _ in range(2)],
        [pltpu.VMEM((CR, H), jnp.float32) for _ in range(3)],
        [pltpu.SemaphoreType.DMA for _ in range(3)],
        [pltpu.SemaphoreType.DMA for _ in range(3)],
        pltpu.SemaphoreType.DMA,
    ],
)


def kernel(edge_index, edge_type, emb, basis1, comp1, root1, bias1,
           basis2, comp2, root2, bias2):
    src = edge_index[0].astype(jnp.int32)
    dst = edge_index[1].astype(jnp.int32)
    et = edge_type.astype(jnp.int32)
    pad = EP - E
    key1 = jnp.concatenate([et * N + src, jnp.zeros((pad,), jnp.int32)])
    key2 = jnp.concatenate([et * N + dst, jnp.full((pad,), RN, jnp.int32)])
    dstp = jnp.concatenate([dst, jnp.full((pad,), N, jnp.int32)])
    key1 = key1.reshape(NW, CH, CR)
    key2 = key2.reshape(NW, CH, CR)
    dstp = dstp.reshape(NW, CH, CR)

    cnt = _count_call(key2)
    w = _weights_call(cnt[:CNTP], cnt[CNTP:], key2)

    key1g = key1.reshape(NW, NG, G, CR)
    dstg = dstp.reshape(NW, NG, G, CR)
    wg = w.reshape(NW, NG, G, CR)

    y1 = _transform(comp1, basis1, emb)
    p1 = _agg_call(y1.reshape(RN, H), key1g, dstg, wg)
    x1 = _combine(emb, root1, bias1.reshape(1, H), p1, True)

    y2 = _transform(comp2, basis2, x1)
    p2 = _agg_call(y2.reshape(RN, H), key1g, dstg, wg)
    out = _combine(x1, root2, bias2.reshape(1, H), p2, False)
    return out


# speedup vs baseline: 38.2224x; 1.0547x over previous
import functools

import jax
import jax.numpy as jnp
from jax import lax
from jax.experimental import pallas as pl
from jax.experimental.pallas import tpu as pltpu
from jax.experimental.pallas import tpu_sc as plsc

N = 10000
R = 16
NBASES = 16
H = 128
E = 320000
NC, NS, LANES = 2, 16, 16
NW = NC * NS
CR = 80
CH = 126
G = 21
NG = CH // G
EPT = CH * CR
EP = NW * EPT
RN = R * N
CNT_PER_TILE = 10112
CNTP = NS * CNT_PER_TILE
NP = 10112
TPR = NP // NS
NZC = TPR // CR
BN = 1000


def _sc_count(key2_hbm, cnt_out, cnt_sh, key2_v, zeros_v, ones_v):
    cid = lax.axis_index("c")
    sid = lax.axis_index("s")
    wid = cid * NS + sid
    zv = jnp.zeros((LANES,), jnp.float32)

    def zfill(i, _):
        zeros_v[pl.ds(i * LANES, LANES)] = zv
        return 0

    lax.fori_loop(0, CNT_PER_TILE // LANES, zfill, 0)
    for k in range(CR // LANES):
        ones_v[pl.ds(k * LANES, LANES)] = jnp.ones((LANES,), jnp.float32)
    pltpu.sync_copy(zeros_v, cnt_sh.at[pl.ds(sid * CNT_PER_TILE, CNT_PER_TILE)])
    pltpu.sync_copy(key2_hbm.at[wid], key2_v)
    plsc.subcore_barrier()

    def body(j, _):
        pltpu.sync_copy(ones_v, cnt_sh.at[key2_v.at[j]], add=True)
        return 0

    lax.fori_loop(0, CH, body, 0)
    plsc.subcore_barrier()
    pltpu.sync_copy(
        cnt_sh.at[pl.ds(sid * CNT_PER_TILE, CNT_PER_TILE)],
        cnt_out.at[pl.ds(cid * CNTP + sid * CNT_PER_TILE, CNT_PER_TILE)],
    )


def _sc_weights(wtab_hbm, key2_hbm, w_out, key2_v, w_v, wsem):
    cid = lax.axis_index("c")
    sid = lax.axis_index("s")
    wid = cid * NS + sid
    pltpu.sync_copy(key2_hbm.at[wid], key2_v)

    def fire(j, _):
        pltpu.async_copy(wtab_hbm.at[key2_v.at[j]], w_v.at[j], wsem)
        return 0

    lax.fori_loop(0, CH, fire, 0)

    def drain(j, _):
        pltpu.make_async_copy(wtab_hbm.at[key2_v.at[j]], w_v.at[j], wsem).wait()
        return 0

    lax.fori_loop(0, CH, drain, 0)
    pltpu.sync_copy(w_v, w_out.at[wid])


def _sc_agg(y_hbm, key1_hbm, dst_hbm, w_hbm, part_out,
            acc_sh, k1buf, dstbuf, wbuf, rows, gsems, ssems, ksem):
    cid = lax.axis_index("c")
    sid = lax.axis_index("s")
    wid = cid * NS + sid
    zv = jnp.zeros((LANES,), jnp.float32)

    def zfill(i, _):
        for k in range(H // LANES):
            rows[0][i, pl.ds(k * LANES, LANES)] = zv
        return 0

    lax.fori_loop(0, CR, zfill, 0)
    base_r = sid * TPR
    for i in range(NZC):
        pltpu.sync_copy(rows[0], acc_sh.at[pl.ds(base_r + i * CR, CR)])
    rem = TPR - NZC * CR
    pltpu.sync_copy(rows[0].at[pl.ds(0, rem)],
                    acc_sh.at[pl.ds(base_r + NZC * CR, rem)])
    plsc.subcore_barrier()

    def kload(g, p):
        pltpu.async_copy(key1_hbm.at[wid, g], k1buf[p], ksem)
        pltpu.async_copy(dst_hbm.at[wid, g], dstbuf[p], ksem)
        pltpu.async_copy(w_hbm.at[wid, g], wbuf[p], ksem)

    def kwait(p):
        pltpu.make_async_copy(key1_hbm.at[wid, 0], k1buf[p], ksem).wait()
        pltpu.make_async_copy(dst_hbm.at[wid, 0], dstbuf[p], ksem).wait()
        pltpu.make_async_copy(w_hbm.at[wid, 0], wbuf[p], ksem).wait()

    def scale(c, p, buf):
        cvec = jnp.full((LANES,), c, jnp.int32)

        def grp(q, _):
            base = q * 4
            bvec = jnp.full((LANES,), base, jnp.int32)
            for r in range(4):
                wsp = plsc.load_gather(wbuf[p], [cvec, bvec + r])
                row = base + r
                for k in range(H // LANES):
                    sl = pl.ds(k * LANES, LANES)
                    buf[row, sl] = buf[row, sl] * wsp
            return 0

        lax.fori_loop(0, CR // 4, grp, 0)

    def chunk(c, b, p):
        bn = (b + 1) % 3
        pltpu.make_async_copy(rows[bn], acc_sh.at[dstbuf[p].at[c - 2]],
                              ssems[bn]).wait()
        pltpu.async_copy(y_hbm.at[k1buf[p].at[c + 1]], rows[bn], gsems[bn])
        pltpu.make_async_copy(y_hbm.at[k1buf[p].at[c]], rows[b], gsems[b]).wait()
        scale(c, p, rows[b])
        pltpu.async_copy(rows[b], acc_sh.at[dstbuf[p].at[c]], ssems[b], add=True)

    def last_chunk(c, b, p):
        bn = (b + 1) % 3
        pltpu.make_async_copy(rows[bn], acc_sh.at[dstbuf[p].at[c - 2]],
                              ssems[bn]).wait()
        pltpu.make_async_copy(y_hbm.at[k1buf[p].at[c]], rows[b], gsems[b]).wait()
        scale(c, p, rows[b])
        pltpu.async_copy(rows[b], acc_sh.at[dstbuf[p].at[c]], ssems[b], add=True)

    kload(0, 0)
    kwait(0)
    pltpu.async_copy(y_hbm.at[k1buf[0].at[0]], rows[0], gsems[0])

    def pair(g2, _):
        for half in range(2):
            g = g2 * 2 + half
            p = half
            for b in range(3):
                bn = (b + 1) % 3
                if half == 0 and b < 2:
                    @pl.when(g2 > 0)
                    def _(bn=bn, b=b):
                        pltpu.make_async_copy(
                            rows[bn], acc_sh.at[dstbuf[p].at[b]], ssems[bn]).wait()
                else:
                    pltpu.make_async_copy(
                        rows[bn], acc_sh.at[dstbuf[p].at[b]], ssems[bn]).wait()
                pltpu.async_copy(y_hbm.at[k1buf[p].at[b + 1]], rows[bn], gsems[bn])
                pltpu.make_async_copy(y_hbm.at[k1buf[p].at[b]], rows[b],
                                      gsems[b]).wait()
                scale(b, p, rows[b])
                pltpu.async_copy(rows[b], acc_sh.at[dstbuf[p].at[b]],
                                 ssems[b], add=True)
            if half == 0:
                kload(g + 1, 1 - p)
            else:
                @pl.when(g2 < NG // 2 - 1)
                def _():
                    kload(g + 1, 1 - p)

            def inner(j3, _):
                for b in range(3):
                    c = j3 * 3 + b
                    chunk(c, b, p)
                return 0

            lax.fori_loop(1, G // 3 - 1, inner, 0)
            cb = G - 3
            chunk(cb, cb % 3, p)
            chunk(cb + 1, (cb + 1) % 3, p)
            last_chunk(G - 1, (G - 1) % 3, p)
            if half == 0:
                kwait(1 - p)
                pltpu.async_copy(y_hbm.at[k1buf[1 - p].at[0]], rows[0], gsems[0])
            else:
                @pl.when(g2 < NG // 2 - 1)
                def _():
                    kwait(1 - p)
                    pltpu.async_copy(y_hbm.at[k1buf[1 - p].at[0]], rows[0],
                                     gsems[0])
        return 0

    lax.fori_loop(0, NG // 2, pair, 0)
    pfin = (NG - 1) % 2
    for c in (G - 2, G - 1):
        b = c % 3
        pltpu.make_async_copy(rows[b], acc_sh.at[dstbuf[pfin].at[c]],
                              ssems[b]).wait()
    plsc.subcore_barrier()
    for i in range(NZC):
        sl = pl.ds(base_r + i * CR, CR)
        pltpu.sync_copy(acc_sh.at[sl], part_out.at[cid, sl])
    sl = pl.ds(base_r + NZC * CR, rem)
    pltpu.sync_copy(acc_sh.at[sl], part_out.at[cid, sl])


def _tc_wtab_body(c0_ref, c1_ref, o_ref):
    o_ref[...] = 1.0 / jnp.maximum(c0_ref[...] + c1_ref[...], 1.0)


def _wtab(cnt):
    c0 = cnt[:CNTP].reshape(CNTP // 128, 128)
    c1 = cnt[CNTP:].reshape(CNTP // 128, 128)
    out = pl.pallas_call(
        _tc_wtab_body,
        out_shape=jax.ShapeDtypeStruct((CNTP // 128, 128), jnp.float32),
    )(c0, c1)
    return out.reshape(CNTP)


def _tc_weights_body(comp_ref, basis_ref, w_ref):
    w_ref[...] = jnp.dot(
        comp_ref[...], basis_ref[...], preferred_element_type=jnp.float32
    )


def _tc_transform_body(w_ref, x_ref, y_ref):
    y_ref[...] = jnp.dot(
        x_ref[...], w_ref[0], preferred_element_type=jnp.float32
    )[None]


def _tc_combine_body(x_ref, root_ref, bias_ref, p_ref, o_ref, *, act):
    t = jnp.dot(x_ref[...], root_ref[...], preferred_element_type=jnp.float32)
    t = t + bias_ref[...] + p_ref[0] + p_ref[1]
    o_ref[...] = jnp.maximum(t, 0.0) if act else t


def _transform(comp, basis, x):
    wmat = pl.pallas_call(
        _tc_weights_body,
        in_specs=[
            pl.BlockSpec((R, NBASES), lambda: (0, 0)),
            pl.BlockSpec((NBASES, H * H), lambda: (0, 0)),
        ],
        out_specs=pl.BlockSpec((R, H * H), lambda: (0, 0)),
        out_shape=jax.ShapeDtypeStruct((R, H * H), jnp.float32),
    )(comp, basis.reshape(NBASES, H * H))
    return pl.pallas_call(
        _tc_transform_body,
        grid=(R, N // BN),
        in_specs=[
            pl.BlockSpec((1, H, H), lambda r, i: (r, 0, 0)),
            pl.BlockSpec((BN, H), lambda r, i: (i, 0)),
        ],
        out_specs=pl.BlockSpec((1, BN, H), lambda r, i: (r, i, 0)),
        out_shape=jax.ShapeDtypeStruct((R, N, H), jnp.float32),
    )(wmat.astype(jnp.bfloat16).reshape(R, H, H), x)


def _combine(x, root, bias, part, act):
    return pl.pallas_call(
        functools.partial(_tc_combine_body, act=act),
        grid=(N // BN,),
        in_specs=[
            pl.BlockSpec((BN, H), lambda i: (i, 0)),
            pl.BlockSpec((H, H), lambda i: (0, 0)),
            pl.BlockSpec((1, H), lambda i: (0, 0)),
            pl.BlockSpec((2, BN, H), lambda i: (0, i, 0)),
        ],
        out_specs=pl.BlockSpec((BN, H), lambda i: (i, 0)),
        out_shape=jax.ShapeDtypeStruct((N, H), jnp.float32),
    )(x, root, bias, part)


_sc_mesh = plsc.VectorSubcoreMesh(core_axis_name="c", subcore_axis_name="s")
_sc_params = pltpu.CompilerParams(needs_layout_passes=False)

_count_call = pl.kernel(
    _sc_count,
    out_type=jax.ShapeDtypeStruct((NC * CNTP,), jnp.float32),
    mesh=_sc_mesh,
    compiler_params=_sc_params,
    scratch_types=[
        pltpu.VMEM_SHARED((CNTP,), jnp.float32),
        pltpu.VMEM((CH, CR), jnp.int32),
        pltpu.VMEM((CNT_PER_TILE,), jnp.float32),
        pltpu.VMEM((CR,), jnp.float32),
    ],
)

_weights_call = pl.kernel(
    _sc_weights,
    out_type=jax.ShapeDtypeStruct((NW, CH, CR), jnp.float32),
    mesh=_sc_mesh,
    compiler_params=_sc_params,
    scratch_types=[
        pltpu.VMEM((CH, CR), jnp.int32),
        pltpu.VMEM((CH, CR), jnp.float32),
        pltpu.SemaphoreType.DMA,
    ],
)

_agg_call = pl.kernel(
    _sc_agg,
    out_type=jax.ShapeDtypeStruct((NC, NP, H), jnp.float32),
    mesh=_sc_mesh,
    compiler_params=_sc_params,
    scratch_types=[
        pltpu.VMEM_SHARED((NP, H), jnp.float32),
        [pltpu.VMEM((G, CR), jnp.int32) for _ in range(2)],
        [pltpu.VMEM((G, CR), jnp.int32) for _ in range(2)],
        [pltpu.VMEM((G, CR), jnp.float32) for _ in range(2)],
        [pltpu.VMEM((CR, H), jnp.float32) for _ in range(3)],
        [pltpu.SemaphoreType.DMA for _ in range(3)],
        [pltpu.SemaphoreType.DMA for _ in range(3)],
        pltpu.SemaphoreType.DMA,
    ],
)


def kernel(edge_index, edge_type, emb, basis1, comp1, root1, bias1,
           basis2, comp2, root2, bias2):
    src = edge_index[0].astype(jnp.int32)
    dst = edge_index[1].astype(jnp.int32)
    et = edge_type.astype(jnp.int32)
    pad = EP - E
    key1 = jnp.concatenate([et * N + src, jnp.zeros((pad,), jnp.int32)])
    key2 = jnp.concatenate([et * N + dst, jnp.full((pad,), RN, jnp.int32)])
    dstp = jnp.concatenate([dst, jnp.full((pad,), N, jnp.int32)])
    key1 = key1.reshape(NW, CH, CR)
    key2 = key2.reshape(NW, CH, CR)
    dstp = dstp.reshape(NW, CH, CR)

    cnt = _count_call(key2)
    w = _weights_call(_wtab(cnt), key2)

    key1g = key1.reshape(NW, NG, G, CR)
    dstg = dstp.reshape(NW, NG, G, CR)
    wg = w.reshape(NW, NG, G, CR)

    embb = emb.astype(jnp.bfloat16)
    y1 = _transform(comp1, basis1, embb)
    p1 = _agg_call(y1.reshape(RN, H), key1g, dstg, wg)
    x1 = _combine(embb, root1.astype(jnp.bfloat16), bias1.reshape(1, H), p1, True)

    x1b = x1.astype(jnp.bfloat16)
    y2 = _transform(comp2, basis2, x1b)
    p2 = _agg_call(y2.reshape(RN, H), key1g, dstg, wg)
    out = _combine(x1b, root2.astype(jnp.bfloat16), bias2.reshape(1, H), p2, False)
    return out
